# Initial kernel scaffold; baseline (speedup 1.0000x reference)
import jax
import jax.numpy as jnp
from jax.experimental import pallas as pl


def kernel(x, edge_index, W_l1, b1, W_r1, W_l2, b2, W_r2):
    raise NotImplementedError("write your pallas kernel here")


# speedup vs baseline: 3.1434x; 3.1434x over previous
import functools

import jax
import jax.numpy as jnp
from jax import lax
from jax.experimental import pallas as pl
from jax.experimental.pallas import tpu as pltpu
from jax.experimental.pallas import tpu_sc as plsc

N = 10000
E = 160000
D = 256
HALF = D // 2

NS = 16
EPT = E // NS
CH = 80
NCHUNK = EPT // CH
RPT = 624
ZB = 208
NZB = RPT // ZB
TAIL = N - NS * RPT

CH2 = 40
EPW = E // (2 * NS)
NCHUNK2 = EPW // CH2


def _sc_agg_body(table, srcs, dsts, zrows, agg_out,
                 src_v, gidx_v, dst_v, rows_v, agg_sh, sem):
    c = lax.axis_index("c")
    s = lax.axis_index("s")
    base_r = s * RPT

    @pl.loop(0, NZB)
    def _(i):
        pltpu.sync_copy(zrows, agg_sh.at[pl.ds(base_r + i * ZB, ZB)])

    @pl.when(s == 0)
    def _():
        pltpu.sync_copy(zrows.at[pl.ds(0, TAIL)],
                        agg_sh.at[pl.ds(NS * RPT, TAIL)])

    plsc.subcore_barrier()

    ebase = s * EPT

    @pl.loop(0, NCHUNK)
    def _(k):
        off = ebase + k * CH
        pltpu.sync_copy(srcs.at[pl.ds(off, CH)], src_v.at[0])
        pltpu.sync_copy(dsts.at[pl.ds(off, CH)], dst_v.at[0])
        for j in range(CH // 16):
            sl = pl.ds(j * 16, 16)
            gidx_v[0, sl] = src_v[0, sl] * 2 + c
        pltpu.async_copy(table.at[gidx_v.at[0]], rows_v, sem).wait()
        pltpu.sync_copy(rows_v, agg_sh.at[dst_v.at[0]], add=True)

    plsc.subcore_barrier()

    @pl.loop(0, NZB)
    def _(i):
        r0 = base_r + i * ZB
        pltpu.sync_copy(agg_sh.at[pl.ds(r0, ZB)], agg_out.at[c, pl.ds(r0, ZB)])

    @pl.when(s == 0)
    def _():
        pltpu.sync_copy(agg_sh.at[pl.ds(NS * RPT, TAIL)],
                        agg_out.at[c, pl.ds(NS * RPT, TAIL)])


_sc_agg = pl.kernel(
    _sc_agg_body,
    out_type=jax.ShapeDtypeStruct((2, N, HALF), jnp.float32),
    mesh=plsc.VectorSubcoreMesh(core_axis_name="c", subcore_axis_name="s"),
    scratch_types=[
        pltpu.VMEM((1, CH), jnp.int32),
        pltpu.VMEM((1, CH), jnp.int32),
        pltpu.VMEM((1, CH), jnp.int32),
        pltpu.VMEM((CH, HALF), jnp.float32),
        pltpu.VMEM_SHARED((N, HALF), jnp.float32),
        pltpu.SemaphoreType.DMA,
    ],
)


def _sc_cnt_body(dsts, zrows, cnt_out, dst_v, ones_v, cnt_sh):
    c = lax.axis_index("c")
    s = lax.axis_index("s")
    base_r = s * RPT

    @pl.loop(0, NZB)
    def _(i):
        pltpu.sync_copy(zrows, cnt_sh.at[pl.ds(base_r + i * ZB, ZB)])

    @pl.when(s == 0)
    def _():
        pltpu.sync_copy(zrows.at[pl.ds(0, TAIL)],
                        cnt_sh.at[pl.ds(NS * RPT, TAIL)])

    @pl.loop(0, CH2)
    def _(i):
        for j in range(HALF // 16):
            ones_v[i, pl.ds(j * 16, 16)] = jnp.ones((16,), jnp.float32)

    plsc.subcore_barrier()

    ebase = (c * NS + s) * EPW

    @pl.loop(0, NCHUNK2)
    def _(k):
        off = ebase + k * CH2
        pltpu.sync_copy(dsts.at[pl.ds(off, CH2)], dst_v.at[0])
        pltpu.sync_copy(ones_v, cnt_sh.at[dst_v.at[0]], add=True)

    plsc.subcore_barrier()

    @pl.loop(0, NZB)
    def _(i):
        r0 = base_r + i * ZB
        pltpu.sync_copy(cnt_sh.at[pl.ds(r0, ZB)], cnt_out.at[c, pl.ds(r0, ZB)])

    @pl.when(s == 0)
    def _():
        pltpu.sync_copy(cnt_sh.at[pl.ds(NS * RPT, TAIL)],
                        cnt_out.at[c, pl.ds(NS * RPT, TAIL)])


_sc_cnt = pl.kernel(
    _sc_cnt_body,
    out_type=jax.ShapeDtypeStruct((2, N, HALF), jnp.float32),
    mesh=plsc.VectorSubcoreMesh(core_axis_name="c", subcore_axis_name="s"),
    scratch_types=[
        pltpu.VMEM((1, CH2), jnp.int32),
        pltpu.VMEM((CH2, HALF), jnp.float32),
        pltpu.VMEM_SHARED((N, HALF), jnp.float32),
    ],
)


def _tc_layer_body(relu, agg_ref, cnt_ref, x_ref, wl_ref, b_ref, wr_ref,
                   o_ref):
    cnt = cnt_ref[0, :, 0:1] + cnt_ref[1, :, 0:1]
    inv = 1.0 / jnp.maximum(cnt, 1.0)
    mean = jnp.concatenate([agg_ref[0], agg_ref[1]], axis=1) * inv
    acc = lax.dot_general(
        mean, wl_ref[...], (((1,), (0,)), ((), ())),
        preferred_element_type=jnp.float32,
        precision=lax.Precision.HIGHEST)
    acc = acc + lax.dot_general(
        x_ref[...], wr_ref[...], (((1,), (0,)), ((), ())),
        preferred_element_type=jnp.float32,
        precision=lax.Precision.HIGHEST)
    acc = acc + b_ref[...]
    if relu:
        acc = jnp.maximum(acc, 0.0)
    o_ref[...] = acc


def _make_tc_layer(relu, rows_per_block=400):
    rb = rows_per_block
    grid = (N // rb,)
    return pl.pallas_call(
        functools.partial(_tc_layer_body, relu),
        grid=grid,
        in_specs=[
            pl.BlockSpec((2, rb, HALF), lambda i: (0, i, 0)),
            pl.BlockSpec((2, rb, HALF), lambda i: (0, i, 0)),
            pl.BlockSpec((rb, D), lambda i: (i, 0)),
            pl.BlockSpec((D, D), lambda i: (0, 0)),
            pl.BlockSpec((1, D), lambda i: (0, 0)),
            pl.BlockSpec((D, D), lambda i: (0, 0)),
        ],
        out_specs=pl.BlockSpec((rb, D), lambda i: (i, 0)),
        out_shape=jax.ShapeDtypeStruct((N, D), jnp.float32),
    )


_tc_layer_relu = _make_tc_layer(relu=True)
_tc_layer_plain = _make_tc_layer(relu=False)


@jax.jit
def kernel(x, edge_index, W_l1, b1, W_r1, W_l2, b2, W_r2):
    src = edge_index[0].astype(jnp.int32)
    dst = edge_index[1].astype(jnp.int32)

    zrows = jnp.zeros((ZB, HALF), jnp.float32)
    agg1 = _sc_agg(x.reshape(2 * N, HALF), src, dst, zrows)
    cnt = _sc_cnt(dst, zrows)
    h = _tc_layer_relu(agg1, cnt, x, W_l1, b1.reshape(1, D), W_r1)
    agg2 = _sc_agg(h.reshape(2 * N, HALF), src, dst, zrows)
    return _tc_layer_plain(agg2, cnt, h, W_l2, b2.reshape(1, D), W_r2)


# speedup vs baseline: 6.1889x; 1.9688x over previous
import functools

import jax
import jax.numpy as jnp
from jax import lax
from jax.experimental import pallas as pl
from jax.experimental.pallas import tpu as pltpu
from jax.experimental.pallas import tpu_sc as plsc

N = 10000
E = 160000
D = 256
HALF = D // 2

NS = 16
EPT = E // NS
CH = 80
NCHUNK = EPT // CH
PH0 = 64
PH1 = NCHUNK - PH0
RPT = 624
ZB = 208
NZB = RPT // ZB
TAIL = N - NS * RPT

CH2 = 40
EPW = E // (2 * NS)
NCHUNK2 = EPW // CH2


def _sc_agg_body(table, src3, dst3, zrows, agg_out,
                 gidx, didx, buf0, buf1, agg_sh, sem0, sem1):
    c = lax.axis_index("c")
    s = lax.axis_index("s")
    base_r = s * RPT

    @pl.loop(0, NZB)
    def _(i):
        pltpu.sync_copy(zrows, agg_sh.at[pl.ds(base_r + i * ZB, ZB)])

    @pl.when(s == 0)
    def _():
        pltpu.sync_copy(zrows.at[pl.ds(0, TAIL)],
                        agg_sh.at[pl.ds(NS * RPT, TAIL)])

    plsc.subcore_barrier()

    for p0, nc in ((0, PH0), (PH0, PH1)):
        pltpu.sync_copy(src3.at[s, pl.ds(p0, nc)], gidx.at[pl.ds(0, nc)])
        pltpu.sync_copy(dst3.at[s, pl.ds(p0, nc)], didx.at[pl.ds(0, nc)])

        @pl.loop(0, nc)
        def _(k):
            for j in range(CH // 16):
                sl = pl.ds(j * 16, 16)
                gidx[k, sl] = gidx[k, sl] * 2 + c

        pltpu.async_copy(table.at[gidx.at[0]], buf0, sem0)
        if nc % 2 == 0:
            nloop = nc // 2 - 1
        else:
            nloop = (nc - 1) // 2

        @pl.loop(0, nloop)
        def _(i):
            j = 2 * i
            pltpu.async_copy(table.at[gidx.at[j + 1]], buf1, sem1)
            pltpu.make_async_copy(table.at[gidx.at[j]], buf0, sem0).wait()
            pltpu.sync_copy(buf0, agg_sh.at[didx.at[j]], add=True)
            pltpu.async_copy(table.at[gidx.at[j + 2]], buf0, sem0)
            pltpu.make_async_copy(table.at[gidx.at[j + 1]], buf1, sem1).wait()
            pltpu.sync_copy(buf1, agg_sh.at[didx.at[j + 1]], add=True)

        if nc % 2 == 0:
            pltpu.async_copy(table.at[gidx.at[nc - 1]], buf1, sem1)
            pltpu.make_async_copy(table.at[gidx.at[nc - 2]], buf0, sem0).wait()
            pltpu.sync_copy(buf0, agg_sh.at[didx.at[nc - 2]], add=True)
            pltpu.make_async_copy(table.at[gidx.at[nc - 1]], buf1, sem1).wait()
            pltpu.sync_copy(buf1, agg_sh.at[didx.at[nc - 1]], add=True)
        else:
            pltpu.make_async_copy(table.at[gidx.at[nc - 1]], buf0, sem0).wait()
            pltpu.sync_copy(buf0, agg_sh.at[didx.at[nc - 1]], add=True)

    plsc.subcore_barrier()

    @pl.loop(0, NZB)
    def _(i):
        r0 = base_r + i * ZB
        pltpu.sync_copy(agg_sh.at[pl.ds(r0, ZB)], agg_out.at[c, pl.ds(r0, ZB)])

    @pl.when(s == 0)
    def _():
        pltpu.sync_copy(agg_sh.at[pl.ds(NS * RPT, TAIL)],
                        agg_out.at[c, pl.ds(NS * RPT, TAIL)])


_sc_agg = pl.kernel(
    _sc_agg_body,
    out_type=jax.ShapeDtypeStruct((2, N, HALF), jnp.float32),
    mesh=plsc.VectorSubcoreMesh(core_axis_name="c", subcore_axis_name="s"),
    scratch_types=[
        pltpu.VMEM((PH0, CH), jnp.int32),
        pltpu.VMEM((PH0, CH), jnp.int32),
        pltpu.VMEM((CH, HALF), jnp.float32),
        pltpu.VMEM((CH, HALF), jnp.float32),
        pltpu.VMEM_SHARED((N, HALF), jnp.float32),
        pltpu.SemaphoreType.DMA,
        pltpu.SemaphoreType.DMA,
    ],
)


def _sc_cnt_body(dst3, zrows, cnt_out, didx, ones_v, cnt_sh):
    c = lax.axis_index("c")
    s = lax.axis_index("s")
    base_r = s * RPT

    pltpu.sync_copy(dst3.at[c * NS + s], didx)

    @pl.loop(0, NZB)
    def _(i):
        pltpu.sync_copy(zrows, cnt_sh.at[pl.ds(base_r + i * ZB, ZB)])

    @pl.when(s == 0)
    def _():
        pltpu.sync_copy(zrows.at[pl.ds(0, TAIL)],
                        cnt_sh.at[pl.ds(NS * RPT, TAIL)])

    @pl.loop(0, CH2)
    def _(i):
        for j in range(HALF // 16):
            ones_v[i, pl.ds(j * 16, 16)] = jnp.ones((16,), jnp.float32)

    plsc.subcore_barrier()

    @pl.loop(0, NCHUNK2)
    def _(k):
        pltpu.sync_copy(ones_v, cnt_sh.at[didx.at[k]], add=True)

    plsc.subcore_barrier()

    @pl.loop(0, NZB)
    def _(i):
        r0 = base_r + i * ZB
        pltpu.sync_copy(cnt_sh.at[pl.ds(r0, ZB)], cnt_out.at[c, pl.ds(r0, ZB)])

    @pl.when(s == 0)
    def _():
        pltpu.sync_copy(cnt_sh.at[pl.ds(NS * RPT, TAIL)],
                        cnt_out.at[c, pl.ds(NS * RPT, TAIL)])


_sc_cnt = pl.kernel(
    _sc_cnt_body,
    out_type=jax.ShapeDtypeStruct((2, N, HALF), jnp.float32),
    mesh=plsc.VectorSubcoreMesh(core_axis_name="c", subcore_axis_name="s"),
    scratch_types=[
        pltpu.VMEM((NCHUNK2, CH2), jnp.int32),
        pltpu.VMEM((CH2, HALF), jnp.float32),
        pltpu.VMEM_SHARED((N, HALF), jnp.float32),
    ],
)


def _tc_matmul_body(x_ref, w_ref, o_ref):
    o_ref[...] = lax.dot_general(
        x_ref[...], w_ref[...], (((1,), (0,)), ((), ())),
        preferred_element_type=jnp.float32,
        precision=lax.Precision.HIGHEST)


def _tc_layer_body(relu, agg_ref, cnt_ref, r_ref, wl_ref, b_ref, o_ref):
    cnt = cnt_ref[0, :, 0:1] + cnt_ref[1, :, 0:1]
    inv = 1.0 / jnp.maximum(cnt, 1.0)
    mean = jnp.concatenate([agg_ref[0], agg_ref[1]], axis=1) * inv
    acc = lax.dot_general(
        mean, wl_ref[...], (((1,), (0,)), ((), ())),
        preferred_element_type=jnp.float32,
        precision=lax.Precision.HIGHEST)
    acc = acc + r_ref[...] + b_ref[...]
    if relu:
        acc = jnp.maximum(acc, 0.0)
    o_ref[...] = acc


RB = 400

_tc_matmul = pl.pallas_call(
    _tc_matmul_body,
    grid=(N // RB,),
    in_specs=[
        pl.BlockSpec((RB, D), lambda i: (i, 0)),
        pl.BlockSpec((D, D), lambda i: (0, 0)),
    ],
    out_specs=pl.BlockSpec((RB, D), lambda i: (i, 0)),
    out_shape=jax.ShapeDtypeStruct((N, D), jnp.float32),
)


def _make_tc_layer(relu):
    return pl.pallas_call(
        functools.partial(_tc_layer_body, relu),
        grid=(N // RB,),
        in_specs=[
            pl.BlockSpec((2, RB, HALF), lambda i: (0, i, 0)),
            pl.BlockSpec((2, RB, HALF), lambda i: (0, i, 0)),
            pl.BlockSpec((RB, D), lambda i: (i, 0)),
            pl.BlockSpec((D, D), lambda i: (0, 0)),
            pl.BlockSpec((1, D), lambda i: (0, 0)),
        ],
        out_specs=pl.BlockSpec((RB, D), lambda i: (i, 0)),
        out_shape=jax.ShapeDtypeStruct((N, D), jnp.float32),
    )


_tc_layer_relu = _make_tc_layer(relu=True)
_tc_layer_plain = _make_tc_layer(relu=False)


@jax.jit
def kernel(x, edge_index, W_l1, b1, W_r1, W_l2, b2, W_r2):
    src = edge_index[0].astype(jnp.int32)
    dst = edge_index[1].astype(jnp.int32)

    src3 = src.reshape(NS, NCHUNK, CH)
    dst3 = dst.reshape(NS, NCHUNK, CH)
    dst3c = dst.reshape(2 * NS, NCHUNK2, CH2)
    zrows = jnp.zeros((ZB, HALF), jnp.float32)

    r1 = _tc_matmul(x, W_r1)
    agg1 = _sc_agg(x.reshape(2 * N, HALF), src3, dst3, zrows)
    cnt = _sc_cnt(dst3c, zrows)
    h = _tc_layer_relu(agg1, cnt, r1, W_l1, b1.reshape(1, D))
    r2 = _tc_matmul(h, W_r2)
    agg2 = _sc_agg(h.reshape(2 * N, HALF), src3, dst3, zrows)
    return _tc_layer_plain(agg2, cnt, r2, W_l2, b2.reshape(1, D))


# speedup vs baseline: 6.7675x; 1.0935x over previous
import functools

import jax
import jax.numpy as jnp
from jax import lax
from jax.experimental import pallas as pl
from jax.experimental.pallas import tpu as pltpu
from jax.experimental.pallas import tpu_sc as plsc

N = 10000
E = 160000
D = 256
HALF = D // 2

NS = 16
EPT = E // NS
CH = 80
NCHUNK = EPT // CH
PH0 = 64
PH1 = NCHUNK - PH0
RPT = 624
ZB = 208
NZB = RPT // ZB
TAIL = N - NS * RPT

CH2 = 40
EPW = E // (2 * NS)
NCHUNK2 = EPW // CH2


def _sc_agg_body(table, src3, dst3, zrows, agg_out,
                 gidx, didx, buf0, buf1, buf2, agg_sh,
                 gsem0, gsem1, gsem2, ssem0, ssem1, ssem2):
    c = lax.axis_index("c")
    s = lax.axis_index("s")
    base_r = s * RPT

    @pl.loop(0, NZB)
    def _(i):
        pltpu.sync_copy(zrows, agg_sh.at[pl.ds(base_r + i * ZB, ZB)])

    @pl.when(s == 0)
    def _():
        pltpu.sync_copy(zrows.at[pl.ds(0, TAIL)],
                        agg_sh.at[pl.ds(NS * RPT, TAIL)])

    plsc.subcore_barrier()

    bufs = (buf0, buf1, buf2)
    gsems = (gsem0, gsem1, gsem2)
    ssems = (ssem0, ssem1, ssem2)

    def g_start(k, b):
        pltpu.async_copy(table.at[gidx.at[k]], bufs[b], gsems[b])

    def g_wait(k, b):
        pltpu.make_async_copy(table.at[gidx.at[k]], bufs[b], gsems[b]).wait()

    def s_start(k, b):
        pltpu.async_copy(bufs[b], agg_sh.at[didx.at[k]], ssems[b], add=True)

    def s_wait(k, b):
        pltpu.make_async_copy(bufs[b], agg_sh.at[didx.at[k]],
                              ssems[b]).wait()

    for p0, nc in ((0, PH0), (PH0, PH1)):
        pltpu.sync_copy(src3.at[s, pl.ds(p0, nc)], gidx.at[pl.ds(0, nc)])
        pltpu.sync_copy(dst3.at[s, pl.ds(p0, nc)], didx.at[pl.ds(0, nc)])

        @pl.loop(0, nc)
        def _(k):
            for j in range(CH // 16):
                sl = pl.ds(j * 16, 16)
                gidx[k, sl] = gidx[k, sl] * 2 + c

        g_start(0, 0)
        g_start(1, 1)
        g_wait(0, 0)
        s_start(0, 0)
        g_start(2, 2)

        T = (nc - 4) // 3
        assert (nc - 4) % 3 == 0

        @pl.loop(0, T)
        def _(t):
            k1 = 3 * t + 1
            for d, b in ((0, 1), (1, 2), (2, 0)):
                k = k1 + d
                g_wait(k, b)
                s_start(k, b)
                s_wait(k - 1, (b + 2) % 3)
                g_start(k + 2, (b + 2) % 3)

        ka = nc - 3
        g_wait(ka, 1)
        s_start(ka, 1)
        s_wait(ka - 1, 0)
        g_start(ka + 2, 0)
        g_wait(ka + 1, 2)
        s_start(ka + 1, 2)
        g_wait(ka + 2, 0)
        s_start(ka + 2, 0)
        s_wait(ka, 1)
        s_wait(ka + 1, 2)
        s_wait(ka + 2, 0)

    plsc.subcore_barrier()

    @pl.loop(0, NZB)
    def _(i):
        r0 = base_r + i * ZB
        pltpu.sync_copy(agg_sh.at[pl.ds(r0, ZB)], agg_out.at[c, pl.ds(r0, ZB)])

    @pl.when(s == 0)
    def _():
        pltpu.sync_copy(agg_sh.at[pl.ds(NS * RPT, TAIL)],
                        agg_out.at[c, pl.ds(NS * RPT, TAIL)])


_sc_agg = pl.kernel(
    _sc_agg_body,
    out_type=jax.ShapeDtypeStruct((2, N, HALF), jnp.float32),
    mesh=plsc.VectorSubcoreMesh(core_axis_name="c", subcore_axis_name="s"),
    scratch_types=[
        pltpu.VMEM((PH0, CH), jnp.int32),
        pltpu.VMEM((PH0, CH), jnp.int32),
        pltpu.VMEM((CH, HALF), jnp.float32),
        pltpu.VMEM((CH, HALF), jnp.float32),
        pltpu.VMEM((CH, HALF), jnp.float32),
        pltpu.VMEM_SHARED((N, HALF), jnp.float32),
        pltpu.SemaphoreType.DMA,
        pltpu.SemaphoreType.DMA,
        pltpu.SemaphoreType.DMA,
        pltpu.SemaphoreType.DMA,
        pltpu.SemaphoreType.DMA,
        pltpu.SemaphoreType.DMA,
    ],
)


def _sc_cnt_body(dst3, zrows, cnt_out, didx, ones_v, cnt_sh):
    c = lax.axis_index("c")
    s = lax.axis_index("s")
    base_r = s * RPT

    pltpu.sync_copy(dst3.at[c * NS + s], didx)

    @pl.loop(0, NZB)
    def _(i):
        pltpu.sync_copy(zrows, cnt_sh.at[pl.ds(base_r + i * ZB, ZB)])

    @pl.when(s == 0)
    def _():
        pltpu.sync_copy(zrows.at[pl.ds(0, TAIL)],
                        cnt_sh.at[pl.ds(NS * RPT, TAIL)])

    @pl.loop(0, CH2)
    def _(i):
        for j in range(HALF // 16):
            ones_v[i, pl.ds(j * 16, 16)] = jnp.ones((16,), jnp.float32)

    plsc.subcore_barrier()

    @pl.loop(0, NCHUNK2)
    def _(k):
        pltpu.sync_copy(ones_v, cnt_sh.at[didx.at[k]], add=True)

    plsc.subcore_barrier()

    @pl.loop(0, NZB)
    def _(i):
        r0 = base_r + i * ZB
        pltpu.sync_copy(cnt_sh.at[pl.ds(r0, ZB)], cnt_out.at[c, pl.ds(r0, ZB)])

    @pl.when(s == 0)
    def _():
        pltpu.sync_copy(cnt_sh.at[pl.ds(NS * RPT, TAIL)],
                        cnt_out.at[c, pl.ds(NS * RPT, TAIL)])


_sc_cnt = pl.kernel(
    _sc_cnt_body,
    out_type=jax.ShapeDtypeStruct((2, N, HALF), jnp.float32),
    mesh=plsc.VectorSubcoreMesh(core_axis_name="c", subcore_axis_name="s"),
    scratch_types=[
        pltpu.VMEM((NCHUNK2, CH2), jnp.int32),
        pltpu.VMEM((CH2, HALF), jnp.float32),
        pltpu.VMEM_SHARED((N, HALF), jnp.float32),
    ],
)


def _tc_matmul_body(x_ref, w_ref, o_ref):
    o_ref[...] = lax.dot_general(
        x_ref[...], w_ref[...], (((1,), (0,)), ((), ())),
        preferred_element_type=jnp.float32,
        precision=lax.Precision.HIGHEST)


def _tc_layer_body(relu, agg_ref, cnt_ref, r_ref, wl_ref, b_ref, o_ref):
    cnt = cnt_ref[0, :, 0:1] + cnt_ref[1, :, 0:1]
    inv = 1.0 / jnp.maximum(cnt, 1.0)
    mean = jnp.concatenate([agg_ref[0], agg_ref[1]], axis=1) * inv
    acc = lax.dot_general(
        mean, wl_ref[...], (((1,), (0,)), ((), ())),
        preferred_element_type=jnp.float32,
        precision=lax.Precision.HIGHEST)
    acc = acc + r_ref[...] + b_ref[...]
    if relu:
        acc = jnp.maximum(acc, 0.0)
    o_ref[...] = acc


RB = 400

_tc_matmul = pl.pallas_call(
    _tc_matmul_body,
    grid=(N // RB,),
    in_specs=[
        pl.BlockSpec((RB, D), lambda i: (i, 0)),
        pl.BlockSpec((D, D), lambda i: (0, 0)),
    ],
    out_specs=pl.BlockSpec((RB, D), lambda i: (i, 0)),
    out_shape=jax.ShapeDtypeStruct((N, D), jnp.float32),
)


def _make_tc_layer(relu):
    return pl.pallas_call(
        functools.partial(_tc_layer_body, relu),
        grid=(N // RB,),
        in_specs=[
            pl.BlockSpec((2, RB, HALF), lambda i: (0, i, 0)),
            pl.BlockSpec((2, RB, HALF), lambda i: (0, i, 0)),
            pl.BlockSpec((RB, D), lambda i: (i, 0)),
            pl.BlockSpec((D, D), lambda i: (0, 0)),
            pl.BlockSpec((1, D), lambda i: (0, 0)),
        ],
        out_specs=pl.BlockSpec((RB, D), lambda i: (i, 0)),
        out_shape=jax.ShapeDtypeStruct((N, D), jnp.float32),
    )


_tc_layer_relu = _make_tc_layer(relu=True)
_tc_layer_plain = _make_tc_layer(relu=False)


@jax.jit
def kernel(x, edge_index, W_l1, b1, W_r1, W_l2, b2, W_r2):
    src = edge_index[0].astype(jnp.int32)
    dst = edge_index[1].astype(jnp.int32)

    src3 = src.reshape(NS, NCHUNK, CH)
    dst3 = dst.reshape(NS, NCHUNK, CH)
    dst3c = dst.reshape(2 * NS, NCHUNK2, CH2)
    zrows = jnp.zeros((ZB, HALF), jnp.float32)

    r1 = _tc_matmul(x, W_r1)
    agg1 = _sc_agg(x.reshape(2 * N, HALF), src3, dst3, zrows)
    cnt = _sc_cnt(dst3c, zrows)
    h = _tc_layer_relu(agg1, cnt, r1, W_l1, b1.reshape(1, D))
    r2 = _tc_matmul(h, W_r2)
    agg2 = _sc_agg(h.reshape(2 * N, HALF), src3, dst3, zrows)
    return _tc_layer_plain(agg2, cnt, r2, W_l2, b2.reshape(1, D))


# speedup vs baseline: 6.8486x; 1.0120x over previous
import functools

import jax
import jax.numpy as jnp
from jax import lax
from jax.experimental import pallas as pl
from jax.experimental.pallas import tpu as pltpu
from jax.experimental.pallas import tpu_sc as plsc

N = 10000
E = 160000
D = 256
HALF = D // 2

NS = 16
EPT = E // NS
CH = 80
NCHUNK = EPT // CH
PH0 = 64
PH1 = NCHUNK - PH0
RPT = 624
ZB = 208
NZB = RPT // ZB
TAIL = N - NS * RPT

CH2 = 40
EPW = E // (2 * NS)
NCHUNK2 = EPW // CH2


def _sc_agg_body(table, src3, dst3, zrows, agg_out,
                 gidx, didx, buf0, buf1, buf2, agg_sh,
                 gsem0, gsem1, gsem2, ssem0, ssem1, ssem2):
    c = lax.axis_index("c")
    s = lax.axis_index("s")
    base_r = s * RPT

    @pl.loop(0, NZB)
    def _(i):
        pltpu.sync_copy(zrows, agg_sh.at[pl.ds(base_r + i * ZB, ZB)])

    @pl.when(s == 0)
    def _():
        pltpu.sync_copy(zrows.at[pl.ds(0, TAIL)],
                        agg_sh.at[pl.ds(NS * RPT, TAIL)])

    plsc.subcore_barrier()

    bufs = (buf0, buf1, buf2)
    gsems = (gsem0, gsem1, gsem2)
    ssems = (ssem0, ssem1, ssem2)

    def g_start(k, b):
        pltpu.async_copy(table.at[gidx.at[k]], bufs[b], gsems[b])

    def g_wait(k, b):
        pltpu.make_async_copy(table.at[gidx.at[k]], bufs[b], gsems[b]).wait()

    def s_start(k, b):
        pltpu.async_copy(bufs[b], agg_sh.at[didx.at[k]], ssems[b], add=True)

    def s_wait(k, b):
        pltpu.make_async_copy(bufs[b], agg_sh.at[didx.at[k]],
                              ssems[b]).wait()

    for p0, nc in ((0, PH0), (PH0, PH1)):
        pltpu.sync_copy(src3.at[s, pl.ds(p0, nc)], gidx.at[pl.ds(0, nc)])
        pltpu.sync_copy(dst3.at[s, pl.ds(p0, nc)], didx.at[pl.ds(0, nc)])

        @pl.loop(0, nc)
        def _(k):
            for j in range(CH // 16):
                sl = pl.ds(j * 16, 16)
                gidx[k, sl] = gidx[k, sl] * 2 + c

        g_start(0, 0)
        g_start(1, 1)
        g_wait(0, 0)
        s_start(0, 0)
        g_start(2, 2)

        T = (nc - 4) // 3
        assert (nc - 4) % 3 == 0

        @pl.loop(0, T)
        def _(t):
            k1 = 3 * t + 1
            for d, b in ((0, 1), (1, 2), (2, 0)):
                k = k1 + d
                g_wait(k, b)
                s_start(k, b)
                s_wait(k - 1, (b + 2) % 3)
                g_start(k + 2, (b + 2) % 3)

        ka = nc - 3
        g_wait(ka, 1)
        s_start(ka, 1)
        s_wait(ka - 1, 0)
        g_start(ka + 2, 0)
        g_wait(ka + 1, 2)
        s_start(ka + 1, 2)
        g_wait(ka + 2, 0)
        s_start(ka + 2, 0)
        s_wait(ka, 1)
        s_wait(ka + 1, 2)
        s_wait(ka + 2, 0)

    plsc.subcore_barrier()

    @pl.loop(0, NZB)
    def _(i):
        r0 = base_r + i * ZB
        pltpu.sync_copy(agg_sh.at[pl.ds(r0, ZB)], agg_out.at[c, pl.ds(r0, ZB)])

    @pl.when(s == 0)
    def _():
        pltpu.sync_copy(agg_sh.at[pl.ds(NS * RPT, TAIL)],
                        agg_out.at[c, pl.ds(NS * RPT, TAIL)])


_sc_agg = pl.kernel(
    _sc_agg_body,
    out_type=jax.ShapeDtypeStruct((2, N, HALF), jnp.float32),
    mesh=plsc.VectorSubcoreMesh(core_axis_name="c", subcore_axis_name="s"),
    scratch_types=[
        pltpu.VMEM((PH0, CH), jnp.int32),
        pltpu.VMEM((PH0, CH), jnp.int32),
        pltpu.VMEM((CH, HALF), jnp.float32),
        pltpu.VMEM((CH, HALF), jnp.float32),
        pltpu.VMEM((CH, HALF), jnp.float32),
        pltpu.VMEM_SHARED((N, HALF), jnp.float32),
        pltpu.SemaphoreType.DMA,
        pltpu.SemaphoreType.DMA,
        pltpu.SemaphoreType.DMA,
        pltpu.SemaphoreType.DMA,
        pltpu.SemaphoreType.DMA,
        pltpu.SemaphoreType.DMA,
    ],
)


def _sc_cnt_body(dst3, zrows, cnt_out, didx, ones_v, cnt_sh,
                 csem0, csem1, csem2):
    c = lax.axis_index("c")
    s = lax.axis_index("s")
    base_r = s * RPT

    pltpu.sync_copy(dst3.at[c * NS + s], didx)

    @pl.loop(0, NZB)
    def _(i):
        pltpu.sync_copy(zrows, cnt_sh.at[pl.ds(base_r + i * ZB, ZB)])

    @pl.when(s == 0)
    def _():
        pltpu.sync_copy(zrows.at[pl.ds(0, TAIL)],
                        cnt_sh.at[pl.ds(NS * RPT, TAIL)])

    @pl.loop(0, CH2)
    def _(i):
        for j in range(HALF // 16):
            ones_v[i, pl.ds(j * 16, 16)] = jnp.ones((16,), jnp.float32)

    plsc.subcore_barrier()

    def cs_start(k, b):
        pltpu.async_copy(ones_v, cnt_sh.at[didx.at[k]], csems[b], add=True)

    def cs_wait(k, b):
        pltpu.make_async_copy(ones_v, cnt_sh.at[didx.at[k]],
                              csems[b]).wait()

    csems = (csem0, csem1, csem2)
    cs_start(0, 0)
    cs_start(1, 1)
    cs_start(2, 2)

    @pl.loop(1, (NCHUNK2 - 2) // 3)
    def _(t):
        for d in range(3):
            k = 3 * t + d
            cs_wait(k - 3, d)
            cs_start(k, d)

    cs_wait(120, 0)
    cs_start(123, 0)
    cs_wait(121, 1)
    cs_start(124, 1)
    cs_wait(122, 2)
    cs_wait(123, 0)
    cs_wait(124, 1)

    plsc.subcore_barrier()

    @pl.loop(0, NZB)
    def _(i):
        r0 = base_r + i * ZB
        pltpu.sync_copy(cnt_sh.at[pl.ds(r0, ZB)], cnt_out.at[c, pl.ds(r0, ZB)])

    @pl.when(s == 0)
    def _():
        pltpu.sync_copy(cnt_sh.at[pl.ds(NS * RPT, TAIL)],
                        cnt_out.at[c, pl.ds(NS * RPT, TAIL)])


_sc_cnt = pl.kernel(
    _sc_cnt_body,
    out_type=jax.ShapeDtypeStruct((2, N, HALF), jnp.float32),
    mesh=plsc.VectorSubcoreMesh(core_axis_name="c", subcore_axis_name="s"),
    scratch_types=[
        pltpu.VMEM((NCHUNK2, CH2), jnp.int32),
        pltpu.VMEM((CH2, HALF), jnp.float32),
        pltpu.VMEM_SHARED((N, HALF), jnp.float32),
        pltpu.SemaphoreType.DMA,
        pltpu.SemaphoreType.DMA,
        pltpu.SemaphoreType.DMA,
    ],
)


def _tc_matmul_body(x_ref, w_ref, o_ref):
    o_ref[...] = lax.dot_general(
        x_ref[...], w_ref[...], (((1,), (0,)), ((), ())),
        preferred_element_type=jnp.float32,
        precision=lax.Precision.HIGHEST)


def _tc_layer_body(relu, agg_ref, cnt_ref, r_ref, wl_ref, b_ref, o_ref):
    cnt = cnt_ref[0, :, 0:1] + cnt_ref[1, :, 0:1]
    inv = 1.0 / jnp.maximum(cnt, 1.0)
    mean = jnp.concatenate([agg_ref[0], agg_ref[1]], axis=1) * inv
    acc = lax.dot_general(
        mean, wl_ref[...], (((1,), (0,)), ((), ())),
        preferred_element_type=jnp.float32,
        precision=lax.Precision.HIGHEST)
    acc = acc + r_ref[...] + b_ref[...]
    if relu:
        acc = jnp.maximum(acc, 0.0)
    o_ref[...] = acc


RB = 400

_tc_matmul = pl.pallas_call(
    _tc_matmul_body,
    grid=(N // RB,),
    in_specs=[
        pl.BlockSpec((RB, D), lambda i: (i, 0)),
        pl.BlockSpec((D, D), lambda i: (0, 0)),
    ],
    out_specs=pl.BlockSpec((RB, D), lambda i: (i, 0)),
    out_shape=jax.ShapeDtypeStruct((N, D), jnp.float32),
)


def _make_tc_layer(relu):
    return pl.pallas_call(
        functools.partial(_tc_layer_body, relu),
        grid=(N // RB,),
        in_specs=[
            pl.BlockSpec((2, RB, HALF), lambda i: (0, i, 0)),
            pl.BlockSpec((2, RB, HALF), lambda i: (0, i, 0)),
            pl.BlockSpec((RB, D), lambda i: (i, 0)),
            pl.BlockSpec((D, D), lambda i: (0, 0)),
            pl.BlockSpec((1, D), lambda i: (0, 0)),
        ],
        out_specs=pl.BlockSpec((RB, D), lambda i: (i, 0)),
        out_shape=jax.ShapeDtypeStruct((N, D), jnp.float32),
    )


_tc_layer_relu = _make_tc_layer(relu=True)
_tc_layer_plain = _make_tc_layer(relu=False)


@jax.jit
def kernel(x, edge_index, W_l1, b1, W_r1, W_l2, b2, W_r2):
    src = edge_index[0].astype(jnp.int32)
    dst = edge_index[1].astype(jnp.int32)

    src3 = src.reshape(NS, NCHUNK, CH)
    dst3 = dst.reshape(NS, NCHUNK, CH)
    dst3c = dst.reshape(2 * NS, NCHUNK2, CH2)
    zrows = jnp.zeros((ZB, HALF), jnp.float32)

    r1 = _tc_matmul(x, W_r1)
    agg1 = _sc_agg(x.reshape(2 * N, HALF), src3, dst3, zrows)
    cnt = _sc_cnt(dst3c, zrows)
    h = _tc_layer_relu(agg1, cnt, r1, W_l1, b1.reshape(1, D))
    r2 = _tc_matmul(h, W_r2)
    agg2 = _sc_agg(h.reshape(2 * N, HALF), src3, dst3, zrows)
    return _tc_layer_plain(agg2, cnt, r2, W_l2, b2.reshape(1, D))


# speedup vs baseline: 7.2037x; 1.0518x over previous
import functools

import jax
import jax.numpy as jnp
from jax import lax
from jax.experimental import pallas as pl
from jax.experimental.pallas import tpu as pltpu
from jax.experimental.pallas import tpu_sc as plsc

N = 10000
E = 160000
D = 256
HALF = D // 2

NS = 16
EPT = E // NS
CH = 80
NCHUNK = EPT // CH
PH0 = 64
PH1 = NCHUNK - PH0
RPT = 624
ZB = 208
NZB = RPT // ZB
TAIL = N - NS * RPT

CH2 = 40
EPW = E // (2 * NS)
NCHUNK2 = EPW // CH2


def _sc_agg_body(table, src3, dst3, zrows, agg_out,
                 gidx, didx, buf0, buf1, buf2, agg_sh,
                 gsem0, gsem1, gsem2, ssem0, ssem1, ssem2):
    c = lax.axis_index("c")
    s = lax.axis_index("s")
    base_r = s * RPT

    @pl.loop(0, NZB)
    def _(i):
        pltpu.sync_copy(zrows, agg_sh.at[pl.ds(base_r + i * ZB, ZB)])

    @pl.when(s == 0)
    def _():
        pltpu.sync_copy(zrows.at[pl.ds(0, TAIL)],
                        agg_sh.at[pl.ds(NS * RPT, TAIL)])

    plsc.subcore_barrier()

    bufs = (buf0, buf1, buf2)
    gsems = (gsem0, gsem1, gsem2)
    ssems = (ssem0, ssem1, ssem2)

    def g_start(k, b):
        pltpu.async_copy(table.at[gidx.at[k]], bufs[b], gsems[b])

    def g_wait(k, b):
        pltpu.make_async_copy(table.at[gidx.at[k]], bufs[b], gsems[b]).wait()

    def s_start(k, b):
        pltpu.async_copy(bufs[b], agg_sh.at[didx.at[k]], ssems[b], add=True)

    def s_wait(k, b):
        pltpu.make_async_copy(bufs[b], agg_sh.at[didx.at[k]],
                              ssems[b]).wait()

    for p0, nc in ((0, PH0), (PH0, PH1)):
        pltpu.sync_copy(src3.at[s, pl.ds(p0, nc)], gidx.at[pl.ds(0, nc)])
        pltpu.sync_copy(dst3.at[s, pl.ds(p0, nc)], didx.at[pl.ds(0, nc)])

        @pl.loop(0, nc)
        def _(k):
            for j in range(CH // 16):
                sl = pl.ds(j * 16, 16)
                gidx[k, sl] = gidx[k, sl] * 2 + c

        g_start(0, 0)
        g_start(1, 1)
        g_wait(0, 0)
        s_start(0, 0)
        g_start(2, 2)

        T = (nc - 4) // 3
        assert (nc - 4) % 3 == 0

        @pl.loop(0, T)
        def _(t):
            k1 = 3 * t + 1
            for d, b in ((0, 1), (1, 2), (2, 0)):
                k = k1 + d
                g_wait(k, b)
                s_start(k, b)
                s_wait(k - 1, (b + 2) % 3)
                g_start(k + 2, (b + 2) % 3)

        ka = nc - 3
        g_wait(ka, 1)
        s_start(ka, 1)
        s_wait(ka - 1, 0)
        g_start(ka + 2, 0)
        g_wait(ka + 1, 2)
        s_start(ka + 1, 2)
        g_wait(ka + 2, 0)
        s_start(ka + 2, 0)
        s_wait(ka, 1)
        s_wait(ka + 1, 2)
        s_wait(ka + 2, 0)

    plsc.subcore_barrier()

    @pl.loop(0, NZB)
    def _(i):
        r0 = base_r + i * ZB
        pltpu.sync_copy(agg_sh.at[pl.ds(r0, ZB)], agg_out.at[c, pl.ds(r0, ZB)])

    @pl.when(s == 0)
    def _():
        pltpu.sync_copy(agg_sh.at[pl.ds(NS * RPT, TAIL)],
                        agg_out.at[c, pl.ds(NS * RPT, TAIL)])


_sc_agg = pl.kernel(
    _sc_agg_body,
    out_type=jax.ShapeDtypeStruct((2, N, HALF), jnp.float32),
    mesh=plsc.VectorSubcoreMesh(core_axis_name="c", subcore_axis_name="s"),
    scratch_types=[
        pltpu.VMEM((PH0, CH), jnp.int32),
        pltpu.VMEM((PH0, CH), jnp.int32),
        pltpu.VMEM((CH, HALF), jnp.float32),
        pltpu.VMEM((CH, HALF), jnp.float32),
        pltpu.VMEM((CH, HALF), jnp.float32),
        pltpu.VMEM_SHARED((N, HALF), jnp.float32),
        pltpu.SemaphoreType.DMA,
        pltpu.SemaphoreType.DMA,
        pltpu.SemaphoreType.DMA,
        pltpu.SemaphoreType.DMA,
        pltpu.SemaphoreType.DMA,
        pltpu.SemaphoreType.DMA,
    ],
)


def _sc_cnt_body(dst3, zrows, cnt_out, didx, ones_v, cnt_sh,
                 csem0, csem1, csem2):
    c = lax.axis_index("c")
    s = lax.axis_index("s")
    base_r = s * RPT

    pltpu.sync_copy(dst3.at[c * NS + s], didx)

    @pl.loop(0, NZB)
    def _(i):
        pltpu.sync_copy(zrows, cnt_sh.at[pl.ds(base_r + i * ZB, ZB)])

    @pl.when(s == 0)
    def _():
        pltpu.sync_copy(zrows.at[pl.ds(0, TAIL)],
                        cnt_sh.at[pl.ds(NS * RPT, TAIL)])

    @pl.loop(0, CH2)
    def _(i):
        for j in range(HALF // 16):
            ones_v[i, pl.ds(j * 16, 16)] = jnp.ones((16,), jnp.float32)

    plsc.subcore_barrier()

    def cs_start(k, b):
        pltpu.async_copy(ones_v, cnt_sh.at[didx.at[k]], csems[b], add=True)

    def cs_wait(k, b):
        pltpu.make_async_copy(ones_v, cnt_sh.at[didx.at[k]],
                              csems[b]).wait()

    csems = (csem0, csem1, csem2)
    cs_start(0, 0)
    cs_start(1, 1)
    cs_start(2, 2)

    @pl.loop(1, (NCHUNK2 - 2) // 3)
    def _(t):
        for d in range(3):
            k = 3 * t + d
            cs_wait(k - 3, d)
            cs_start(k, d)

    cs_wait(120, 0)
    cs_start(123, 0)
    cs_wait(121, 1)
    cs_start(124, 1)
    cs_wait(122, 2)
    cs_wait(123, 0)
    cs_wait(124, 1)

    plsc.subcore_barrier()

    @pl.loop(0, NZB)
    def _(i):
        r0 = base_r + i * ZB
        pltpu.sync_copy(cnt_sh.at[pl.ds(r0, ZB)], cnt_out.at[c, pl.ds(r0, ZB)])

    @pl.when(s == 0)
    def _():
        pltpu.sync_copy(cnt_sh.at[pl.ds(NS * RPT, TAIL)],
                        cnt_out.at[c, pl.ds(NS * RPT, TAIL)])


_sc_cnt = pl.kernel(
    _sc_cnt_body,
    out_type=jax.ShapeDtypeStruct((2, N, HALF), jnp.float32),
    mesh=plsc.VectorSubcoreMesh(core_axis_name="c", subcore_axis_name="s"),
    scratch_types=[
        pltpu.VMEM((NCHUNK2, CH2), jnp.int32),
        pltpu.VMEM((CH2, HALF), jnp.float32),
        pltpu.VMEM_SHARED((N, HALF), jnp.float32),
        pltpu.SemaphoreType.DMA,
        pltpu.SemaphoreType.DMA,
        pltpu.SemaphoreType.DMA,
    ],
)


def _tc_matmul_body(x_ref, w_ref, o_ref):
    o_ref[...] = lax.dot_general(
        x_ref[...], w_ref[...], (((1,), (0,)), ((), ())),
        preferred_element_type=jnp.float32,
        precision=lax.Precision.HIGHEST)


def _tc_layer_body(relu, agg_ref, cnt_ref, r_ref, wl_ref, b_ref, o_ref):
    cnt = cnt_ref[0, :, 0:1] + cnt_ref[1, :, 0:1]
    inv = 1.0 / jnp.maximum(cnt, 1.0)
    mean = jnp.concatenate([agg_ref[0], agg_ref[1]], axis=1) * inv
    acc = lax.dot_general(
        mean, wl_ref[...], (((1,), (0,)), ((), ())),
        preferred_element_type=jnp.float32,
        precision=lax.Precision.HIGHEST)
    acc = acc + r_ref[...] + b_ref[...]
    if relu:
        acc = jnp.maximum(acc, 0.0)
    o_ref[...] = acc


RB = 2000

_tc_matmul = pl.pallas_call(
    _tc_matmul_body,
    grid=(N // RB,),
    in_specs=[
        pl.BlockSpec((RB, D), lambda i: (i, 0)),
        pl.BlockSpec((D, D), lambda i: (0, 0)),
    ],
    out_specs=pl.BlockSpec((RB, D), lambda i: (i, 0)),
    out_shape=jax.ShapeDtypeStruct((N, D), jnp.float32),
)


def _make_tc_layer(relu):
    return pl.pallas_call(
        functools.partial(_tc_layer_body, relu),
        grid=(N // RB,),
        in_specs=[
            pl.BlockSpec((2, RB, HALF), lambda i: (0, i, 0)),
            pl.BlockSpec((2, RB, HALF), lambda i: (0, i, 0)),
            pl.BlockSpec((RB, D), lambda i: (i, 0)),
            pl.BlockSpec((D, D), lambda i: (0, 0)),
            pl.BlockSpec((1, D), lambda i: (0, 0)),
        ],
        out_specs=pl.BlockSpec((RB, D), lambda i: (i, 0)),
        out_shape=jax.ShapeDtypeStruct((N, D), jnp.float32),
    )


_tc_layer_relu = _make_tc_layer(relu=True)
_tc_layer_plain = _make_tc_layer(relu=False)


@jax.jit
def kernel(x, edge_index, W_l1, b1, W_r1, W_l2, b2, W_r2):
    src = edge_index[0].astype(jnp.int32)
    dst = edge_index[1].astype(jnp.int32)

    src3 = src.reshape(NS, NCHUNK, CH)
    dst3 = dst.reshape(NS, NCHUNK, CH)
    dst3c = dst.reshape(2 * NS, NCHUNK2, CH2)
    zrows = jnp.zeros((ZB, HALF), jnp.float32)

    r1 = _tc_matmul(x, W_r1)
    agg1 = _sc_agg(x.reshape(2 * N, HALF), src3, dst3, zrows)
    cnt = _sc_cnt(dst3c, zrows)
    h = _tc_layer_relu(agg1, cnt, r1, W_l1, b1.reshape(1, D))
    r2 = _tc_matmul(h, W_r2)
    agg2 = _sc_agg(h.reshape(2 * N, HALF), src3, dst3, zrows)
    return _tc_layer_plain(agg2, cnt, r2, W_l2, b2.reshape(1, D))
